# baseline (speedup 1.0000x reference)
import functools

import jax
import jax.numpy as jnp
from jax import lax
from jax.experimental import pallas as pl
from jax.experimental.pallas import tpu as pltpu
from jax.experimental.pallas import tpu_sc as plsc

N = 10000
DIN = 128
DH = 128
C = 40
CP = 64
NC = 2
NS = 16
NW = NC * NS
B = 128
NP = 10240
RT = NP // NS
RB = 1000
f32 = jnp.float32

@functools.lru_cache(maxsize=None)
def _mesh():
    return plsc.VectorSubcoreMesh(
        core_axis_name="c", subcore_axis_name="s", num_cores=NC, num_subcores=NS
    )


def _deg_body(sdp, ones8, zeros8, out, idxr, ones_v, dacc, s_i, s_s, *, cpt):
    c = lax.axis_index("c")
    s = lax.axis_index("s")
    w = s * NC + c
    r0 = pl.multiple_of(s * RT, 8)
    pltpu.sync_copy(ones8, ones_v)
    pltpu.sync_copy(zeros8, dacc.at[pl.ds(r0, RT)])
    plsc.subcore_barrier()

    def wait_scatter(t):
        pltpu.make_async_copy(ones_v, dacc.at[idxr.at[lax.rem(t, 5)]], s_s).wait()

    def body(t, carry):
        @pl.when(jnp.logical_and(t >= 5, t - 5 < cpt))
        def _():
            wait_scatter(t - 5)

        @pl.when(t < cpt)
        def _():
            base = pl.multiple_of((w * cpt + t) * B, B)
            pltpu.async_copy(sdp.at[1, pl.ds(base, B)], idxr.at[lax.rem(t, 5)], s_i)

        @pl.when(jnp.logical_and(t >= 1, t - 1 < cpt))
        def _():
            bi = lax.rem(t - 1, 5)
            pltpu.make_async_copy(sdp.at[1, pl.ds(0, B)], idxr.at[bi], s_i).wait()
            pltpu.async_copy(ones_v, dacc.at[idxr.at[bi]], s_s, add=True)

        return carry

    lax.fori_loop(0, cpt + 1, body, 0)

    def drain(t, carry):
        wait_scatter(t)
        return carry

    lax.fori_loop(cpt + 1 - 5, cpt, drain, 0)
    plsc.subcore_barrier()
    pltpu.sync_copy(dacc.at[pl.ds(r0, RT)], out.at[c, pl.ds(r0, RT)])


def _make_deg(cpt):
    return pl.kernel(
        functools.partial(_deg_body, cpt=cpt),
        out_type=jax.ShapeDtypeStruct((NC, NP, 8), f32),
        mesh=_mesh(),
        scratch_types=[
            pltpu.VMEM((5, B), jnp.int32),
            pltpu.VMEM((B, 8), f32),
            pltpu.VMEM_SHARED((NP, 8), f32),
            pltpu.SemaphoreType.DMA,
            pltpu.SemaphoreType.DMA,
        ],
        compiler_params=pltpu.CompilerParams(use_tc_tiling_on_sc=False),
    )


def _agg_body(sdp, hp, out, idx2, rows, tab, acc, s_i, s_g, s_s, *, d, cpt2, nrb, nib):
    bf16 = jnp.bfloat16
    c = lax.axis_index("c")
    s = lax.axis_index("s")
    r0 = pl.multiple_of(s * RT, 8)
    kpr = d // 32
    col0 = c * d

    def zv(i, carry):
        r = i // kpr
        k = lax.rem(i, kpr)
        rows[0, r, pl.ds(pl.multiple_of(k * 32, 32), 32)] = jnp.zeros((32,), bf16)
        return carry

    lax.fori_loop(0, B * kpr, zv, 0)

    def zc(j, carry):
        pltpu.sync_copy(rows.at[0], acc.at[pl.ds(pl.multiple_of(r0 + j * B, 8), B)])
        return carry

    lax.fori_loop(0, RT // B, zc, 0)

    @pl.when(s < NS - 1)
    def _():
        b0 = pl.multiple_of(s * 640, 8)
        pltpu.sync_copy(hp.at[pl.ds(b0, 640), pl.ds(col0, d)], tab.at[pl.ds(b0, 640)])

    @pl.when(s == NS - 1)
    def _():
        pltpu.sync_copy(hp.at[pl.ds(9600, N - 9600), pl.ds(col0, d)],
                        tab.at[pl.ds(9600, N - 9600)])

    plsc.subcore_barrier()

    def wait_scatter(t):
        b = lax.rem(t, nrb)
        bi = lax.rem(t, nib)
        pltpu.make_async_copy(rows.at[b], acc.at[idx2.at[bi, 1]], s_s).wait()

    def body(t, carry):
        @pl.when(jnp.logical_and(t >= nib, t - nib < cpt2))
        def _():
            wait_scatter(t - nib)

        @pl.when(t < cpt2)
        def _():
            base = pl.multiple_of((s * cpt2 + t) * B, B)
            pltpu.async_copy(sdp.at[:, pl.ds(base, B)], idx2.at[lax.rem(t, nib)], s_i)

        @pl.when(jnp.logical_and(t >= 1, t - 1 < cpt2))
        def _():
            b = lax.rem(t - 1, nrb)
            bi = lax.rem(t - 1, nib)
            pltpu.make_async_copy(sdp.at[:, pl.ds(0, B)], idx2.at[bi], s_i).wait()
            pltpu.async_copy(tab.at[idx2.at[bi, 0]], rows.at[b], s_g)

        @pl.when(jnp.logical_and(t >= 2, t - 2 < cpt2))
        def _():
            b = lax.rem(t - 2, nrb)
            bi = lax.rem(t - 2, nib)
            pltpu.make_async_copy(tab.at[idx2.at[bi, 0]], rows.at[b], s_g).wait()
            pltpu.async_copy(rows.at[b], acc.at[idx2.at[bi, 1]], s_s, add=True)

        return carry

    lax.fori_loop(0, cpt2 + 2, body, 0)

    def drain(t, carry):
        wait_scatter(t)
        return carry

    lax.fori_loop(cpt2 + 2 - nib, cpt2, drain, 0)
    plsc.subcore_barrier()
    pltpu.sync_copy(acc.at[pl.ds(r0, RT)], out.at[c, pl.ds(r0, RT)])


def _make_agg(d2, cpt2):
    d = d2 // 2
    nrb, nib = (4, 6) if d == 64 else (6, 8)
    return pl.kernel(
        functools.partial(_agg_body, d=d, cpt2=cpt2, nrb=nrb, nib=nib),
        out_type=jax.ShapeDtypeStruct((NC, NP, d), jnp.bfloat16),
        mesh=_mesh(),
        scratch_types=[
            pltpu.VMEM((nib, 2, B), jnp.int32),
            pltpu.VMEM((nrb, B, d), jnp.bfloat16),
            pltpu.VMEM_SHARED((N, d), jnp.bfloat16),
            pltpu.VMEM_SHARED((NP, d), jnp.bfloat16),
            pltpu.SemaphoreType.DMA,
            pltpu.SemaphoreType.DMA,
            pltpu.SemaphoreType.DMA,
        ],
        compiler_params=pltpu.CompilerParams(use_tc_tiling_on_sc=False),
    )


def _p1a_body(x_ref, w1_ref, h_ref):
    h_ref[...] = jnp.dot(x_ref[...], w1_ref[...], preferred_element_type=f32)


_p1a = pl.pallas_call(
    _p1a_body,
    grid=(N // RB,),
    in_specs=[
        pl.BlockSpec((RB, DIN), lambda i: (i, 0)),
        pl.BlockSpec((DIN, DH), lambda i: (0, 0)),
    ],
    out_specs=pl.BlockSpec((RB, DH), lambda i: (i, 0)),
    out_shape=jax.ShapeDtypeStruct((N, DH), f32),
)


def _p1b_body(h_ref, dc_ref, hp_ref, dis_ref):
    cnt = dc_ref[0] + dc_ref[1]
    deg = jnp.sum(cnt, axis=1, keepdims=True) * 0.125 + 1.0
    dis = lax.rsqrt(deg)
    hp_ref[...] = (h_ref[...] * dis).astype(hp_ref.dtype)
    dis_ref[...] = dis


_p1b = pl.pallas_call(
    _p1b_body,
    grid=(N // RB,),
    in_specs=[
        pl.BlockSpec((RB, DH), lambda i: (i, 0)),
        pl.BlockSpec((NC, RB, 8), lambda i: (0, i, 0)),
    ],
    out_specs=[
        pl.BlockSpec((RB, DH), lambda i: (i, 0)),
        pl.BlockSpec((RB, 1), lambda i: (i, 0)),
    ],
    out_shape=[
        jax.ShapeDtypeStruct((N, DH), jnp.bfloat16),
        jax.ShapeDtypeStruct((N, 1), f32),
    ],
)


def _p3_body(a_ref, hp_ref, dis_ref, b1_ref, w2_ref, hp2_ref):
    agg = jnp.concatenate([a_ref[0], a_ref[1]], axis=1).astype(f32)
    dis = dis_ref[...]
    z = dis * (agg + hp_ref[...].astype(f32)) + b1_ref[...]
    z = jnp.maximum(z, 0.0)
    h2 = jnp.dot(z, w2_ref[...], preferred_element_type=f32)
    hp2_ref[...] = (h2 * dis).astype(hp2_ref.dtype)


_p3 = pl.pallas_call(
    _p3_body,
    grid=(N // RB,),
    in_specs=[
        pl.BlockSpec((NC, RB, DH // 2), lambda i: (0, i, 0)),
        pl.BlockSpec((RB, DH), lambda i: (i, 0)),
        pl.BlockSpec((RB, 1), lambda i: (i, 0)),
        pl.BlockSpec((1, DH), lambda i: (0, 0)),
        pl.BlockSpec((DH, CP), lambda i: (0, 0)),
    ],
    out_specs=pl.BlockSpec((RB, CP), lambda i: (i, 0)),
    out_shape=jax.ShapeDtypeStruct((N, CP), jnp.bfloat16),
)


def _p5_body(a_ref, hp2_ref, dis_ref, b2_ref, o_ref):
    agg = jnp.concatenate([a_ref[0], a_ref[1]], axis=1).astype(f32)
    o = dis_ref[...] * (agg + hp2_ref[...].astype(f32)) + b2_ref[...]
    colm = lax.broadcasted_iota(jnp.int32, (RB, CP), 1) < C
    om = jnp.where(colm, o, -1e30)
    m = jnp.max(om, axis=1, keepdims=True)
    e = jnp.where(colm, jnp.exp(o - m), 0.0)
    lse = jnp.log(jnp.sum(e, axis=1, keepdims=True))
    r = o - m - lse
    o_ref[...] = r[:, :C]


_p5 = pl.pallas_call(
    _p5_body,
    grid=(N // RB,),
    in_specs=[
        pl.BlockSpec((NC, RB, CP // 2), lambda i: (0, i, 0)),
        pl.BlockSpec((RB, CP), lambda i: (i, 0)),
        pl.BlockSpec((RB, 1), lambda i: (i, 0)),
        pl.BlockSpec((1, CP), lambda i: (0, 0)),
    ],
    out_specs=pl.BlockSpec((RB, C), lambda i: (i, 0)),
    out_shape=jax.ShapeDtypeStruct((N, C), f32),
)


def kernel(x, edge_index, W1, b1, W2, b2):
    e = edge_index.shape[1]
    nch = -(-e // B)
    cpt = -(-nch // NW)
    ep = cpt * NW * B
    src = edge_index[0]
    dst = edge_index[1]
    srcp = jnp.concatenate([src, jnp.zeros((ep - e,), jnp.int32)])
    dstp = jnp.concatenate([dst, jnp.full((ep - e,), N, jnp.int32)])
    sdp = jnp.stack([srcp, dstp])
    ones8 = jnp.ones((B, 8), f32)
    zeros8 = jnp.zeros((RT, 8), f32)

    cpt2 = ep // (NS * B)
    dcnt = _make_deg(cpt)(sdp, ones8, zeros8)
    h1 = _p1a(x, W1)
    hp1, dis = _p1b(h1, dcnt)
    acc1 = _make_agg(DH, cpt2)(sdp, hp1)
    b1r = b1.reshape(1, DH)
    w2p = jnp.pad(W2, ((0, 0), (0, CP - C)))
    b2r = jnp.pad(b2, (0, CP - C)).reshape(1, CP)
    hp2 = _p3(acc1, hp1, dis, b1r, w2p)
    acc2 = _make_agg(CP, cpt2)(sdp, hp2)
    return _p5(acc2, hp2, dis, b2r)


# speedup vs baseline: 1.0097x; 1.0097x over previous
import functools

import jax
import jax.numpy as jnp
from jax import lax
from jax.experimental import pallas as pl
from jax.experimental.pallas import tpu as pltpu
from jax.experimental.pallas import tpu_sc as plsc

N = 10000
DIN = 128
DH = 128
C = 40
CP = 64
NC = 2
NS = 16
NW = NC * NS
B = 128
NP = 10240
RT = NP // NS
RB = 1000
f32 = jnp.float32

@functools.lru_cache(maxsize=None)
def _mesh():
    return plsc.VectorSubcoreMesh(
        core_axis_name="c", subcore_axis_name="s", num_cores=NC, num_subcores=NS
    )


def _deg_body(sdp, ones8, zeros8, out, idxr, ones_v, dacc, s_i, s_s, *, cpt):
    c = lax.axis_index("c")
    s = lax.axis_index("s")
    w = s * NC + c
    r0 = pl.multiple_of(s * RT, 8)
    pltpu.sync_copy(ones8, ones_v)
    pltpu.sync_copy(zeros8, dacc.at[pl.ds(r0, RT)])
    plsc.subcore_barrier()

    def wait_scatter(t):
        pltpu.make_async_copy(ones_v, dacc.at[idxr.at[lax.rem(t, 5)]], s_s).wait()

    def body(t, carry):
        @pl.when(jnp.logical_and(t >= 5, t - 5 < cpt))
        def _():
            wait_scatter(t - 5)

        @pl.when(t < cpt)
        def _():
            base = pl.multiple_of((w * cpt + t) * B, B)
            pltpu.async_copy(sdp.at[1, pl.ds(base, B)], idxr.at[lax.rem(t, 5)], s_i)

        @pl.when(jnp.logical_and(t >= 1, t - 1 < cpt))
        def _():
            bi = lax.rem(t - 1, 5)
            pltpu.make_async_copy(sdp.at[1, pl.ds(0, B)], idxr.at[bi], s_i).wait()
            pltpu.async_copy(ones_v, dacc.at[idxr.at[bi]], s_s, add=True)

        return carry

    lax.fori_loop(0, cpt + 1, body, 0)

    def drain(t, carry):
        wait_scatter(t)
        return carry

    lax.fori_loop(cpt + 1 - 5, cpt, drain, 0)
    plsc.subcore_barrier()
    pltpu.sync_copy(dacc.at[pl.ds(r0, RT)], out.at[c, pl.ds(r0, RT)])


def _make_deg(cpt):
    return pl.kernel(
        functools.partial(_deg_body, cpt=cpt),
        out_type=jax.ShapeDtypeStruct((NC, NP, 8), f32),
        mesh=_mesh(),
        scratch_types=[
            pltpu.VMEM((5, B), jnp.int32),
            pltpu.VMEM((B, 8), f32),
            pltpu.VMEM_SHARED((NP, 8), f32),
            pltpu.SemaphoreType.DMA,
            pltpu.SemaphoreType.DMA,
        ],
        compiler_params=pltpu.CompilerParams(use_tc_tiling_on_sc=False),
    )


def _agg_body(sdp, hp, out, idx2, rows, tab, acc, s_i, s_g, s_s, *, d, cpt2, nrb, nib):
    bf16 = jnp.bfloat16
    c = lax.axis_index("c")
    s = lax.axis_index("s")
    r0 = pl.multiple_of(s * RT, 8)
    kpr = d // 32
    col0 = c * d

    def zv(i, carry):
        r = i // kpr
        k = lax.rem(i, kpr)
        rows[0, r, pl.ds(pl.multiple_of(k * 32, 32), 32)] = jnp.zeros((32,), bf16)
        return carry

    lax.fori_loop(0, B * kpr, zv, 0)

    def zc(j, carry):
        pltpu.sync_copy(rows.at[0], acc.at[pl.ds(pl.multiple_of(r0 + j * B, 8), B)])
        return carry

    lax.fori_loop(0, RT // B, zc, 0)

    @pl.when(s < NS - 1)
    def _():
        b0 = pl.multiple_of(s * 640, 8)
        pltpu.sync_copy(hp.at[pl.ds(b0, 640), pl.ds(col0, d)], tab.at[pl.ds(b0, 640)])

    @pl.when(s == NS - 1)
    def _():
        pltpu.sync_copy(hp.at[pl.ds(9600, N - 9600), pl.ds(col0, d)],
                        tab.at[pl.ds(9600, N - 9600)])

    plsc.subcore_barrier()

    def wait_scatter(t):
        b = lax.rem(t, nrb)
        bi = lax.rem(t, nib)
        pltpu.make_async_copy(rows.at[b], acc.at[idx2.at[bi, 1]], s_s).wait()

    def body(t, carry):
        @pl.when(jnp.logical_and(t >= nib, t - nib < cpt2))
        def _():
            wait_scatter(t - nib)

        @pl.when(t < cpt2)
        def _():
            base = pl.multiple_of((s * cpt2 + t) * B, B)
            pltpu.async_copy(sdp.at[:, pl.ds(base, B)], idx2.at[lax.rem(t, nib)], s_i)

        @pl.when(jnp.logical_and(t >= 1, t - 1 < cpt2))
        def _():
            b = lax.rem(t - 1, nrb)
            bi = lax.rem(t - 1, nib)
            pltpu.make_async_copy(sdp.at[:, pl.ds(0, B)], idx2.at[bi], s_i).wait()
            pltpu.async_copy(tab.at[idx2.at[bi, 0]], rows.at[b], s_g)

        @pl.when(jnp.logical_and(t >= 2, t - 2 < cpt2))
        def _():
            b = lax.rem(t - 2, nrb)
            bi = lax.rem(t - 2, nib)
            pltpu.make_async_copy(tab.at[idx2.at[bi, 0]], rows.at[b], s_g).wait()
            pltpu.async_copy(rows.at[b], acc.at[idx2.at[bi, 1]], s_s, add=True)

        return carry

    lax.fori_loop(0, cpt2 + 2, body, 0)

    def drain(t, carry):
        wait_scatter(t)
        return carry

    lax.fori_loop(cpt2 + 2 - nib, cpt2, drain, 0)
    plsc.subcore_barrier()
    pltpu.sync_copy(acc.at[pl.ds(r0, RT)], out.at[c, pl.ds(r0, RT)])


def _make_agg(d2, cpt2):
    d = d2 // 2
    nrb, nib = (4, 6) if d == 64 else (6, 8)
    return pl.kernel(
        functools.partial(_agg_body, d=d, cpt2=cpt2, nrb=nrb, nib=nib),
        out_type=jax.ShapeDtypeStruct((NC, NP, d), jnp.bfloat16),
        mesh=_mesh(),
        scratch_types=[
            pltpu.VMEM((nib, 2, B), jnp.int32),
            pltpu.VMEM((nrb, B, d), jnp.bfloat16),
            pltpu.VMEM_SHARED((N, d), jnp.bfloat16),
            pltpu.VMEM_SHARED((NP, d), jnp.bfloat16),
            pltpu.SemaphoreType.DMA,
            pltpu.SemaphoreType.DMA,
            pltpu.SemaphoreType.DMA,
        ],
        compiler_params=pltpu.CompilerParams(use_tc_tiling_on_sc=False),
    )


def _p1_body(x_ref, w1_ref, dc_ref, hp_ref, dis_ref):
    cnt = dc_ref[0] + dc_ref[1]
    deg = jnp.sum(cnt, axis=1, keepdims=True) * 0.125 + 1.0
    dis = lax.rsqrt(deg)
    h = jnp.dot(x_ref[...], w1_ref[...], preferred_element_type=f32)
    hp_ref[...] = (h * dis).astype(hp_ref.dtype)
    dis_ref[...] = dis


_p1 = pl.pallas_call(
    _p1_body,
    grid=(N // RB,),
    in_specs=[
        pl.BlockSpec((RB, DIN), lambda i: (i, 0)),
        pl.BlockSpec((DIN, DH), lambda i: (0, 0)),
        pl.BlockSpec((NC, RB, 8), lambda i: (0, i, 0)),
    ],
    out_specs=[
        pl.BlockSpec((RB, DH), lambda i: (i, 0)),
        pl.BlockSpec((RB, 1), lambda i: (i, 0)),
    ],
    out_shape=[
        jax.ShapeDtypeStruct((N, DH), jnp.bfloat16),
        jax.ShapeDtypeStruct((N, 1), f32),
    ],
)


def _p3_body(a_ref, hp_ref, dis_ref, b1_ref, w2_ref, hp2_ref):
    agg = jnp.concatenate([a_ref[0], a_ref[1]], axis=1).astype(f32)
    dis = dis_ref[...]
    z = dis * (agg + hp_ref[...].astype(f32)) + b1_ref[...]
    z = jnp.maximum(z, 0.0)
    h2 = jnp.dot(z, w2_ref[...], preferred_element_type=f32)
    hp2_ref[...] = (h2 * dis).astype(hp2_ref.dtype)


_p3 = pl.pallas_call(
    _p3_body,
    grid=(N // RB,),
    in_specs=[
        pl.BlockSpec((NC, RB, DH // 2), lambda i: (0, i, 0)),
        pl.BlockSpec((RB, DH), lambda i: (i, 0)),
        pl.BlockSpec((RB, 1), lambda i: (i, 0)),
        pl.BlockSpec((1, DH), lambda i: (0, 0)),
        pl.BlockSpec((DH, CP), lambda i: (0, 0)),
    ],
    out_specs=pl.BlockSpec((RB, CP), lambda i: (i, 0)),
    out_shape=jax.ShapeDtypeStruct((N, CP), jnp.bfloat16),
)


def _p5_body(a_ref, hp2_ref, dis_ref, b2_ref, o_ref):
    agg = jnp.concatenate([a_ref[0], a_ref[1]], axis=1).astype(f32)
    o = dis_ref[...] * (agg + hp2_ref[...].astype(f32)) + b2_ref[...]
    colm = lax.broadcasted_iota(jnp.int32, (RB, CP), 1) < C
    om = jnp.where(colm, o, -1e30)
    m = jnp.max(om, axis=1, keepdims=True)
    e = jnp.where(colm, jnp.exp(o - m), 0.0)
    lse = jnp.log(jnp.sum(e, axis=1, keepdims=True))
    r = o - m - lse
    o_ref[...] = r[:, :C]


_p5 = pl.pallas_call(
    _p5_body,
    grid=(N // RB,),
    in_specs=[
        pl.BlockSpec((NC, RB, CP // 2), lambda i: (0, i, 0)),
        pl.BlockSpec((RB, CP), lambda i: (i, 0)),
        pl.BlockSpec((RB, 1), lambda i: (i, 0)),
        pl.BlockSpec((1, CP), lambda i: (0, 0)),
    ],
    out_specs=pl.BlockSpec((RB, C), lambda i: (i, 0)),
    out_shape=jax.ShapeDtypeStruct((N, C), f32),
)


def kernel(x, edge_index, W1, b1, W2, b2):
    e = edge_index.shape[1]
    nch = -(-e // B)
    cpt = -(-nch // NW)
    ep = cpt * NW * B
    src = edge_index[0]
    dst = edge_index[1]
    srcp = jnp.concatenate([src, jnp.zeros((ep - e,), jnp.int32)])
    dstp = jnp.concatenate([dst, jnp.full((ep - e,), N, jnp.int32)])
    sdp = jnp.stack([srcp, dstp])
    ones8 = jnp.ones((B, 8), f32)
    zeros8 = jnp.zeros((RT, 8), f32)

    cpt2 = ep // (NS * B)
    dcnt = _make_deg(cpt)(sdp, ones8, zeros8)
    hp1, dis = _p1(x, W1, dcnt)
    acc1 = _make_agg(DH, cpt2)(sdp, hp1)
    b1r = b1.reshape(1, DH)
    w2p = jnp.pad(W2, ((0, 0), (0, CP - C)))
    b2r = jnp.pad(b2, (0, CP - C)).reshape(1, CP)
    hp2 = _p3(acc1, hp1, dis, b1r, w2p)
    acc2 = _make_agg(CP, cpt2)(sdp, hp2)
    return _p5(acc2, hp2, dis, b2r)
